# baseline (speedup 1.0000x reference)
import functools

import jax
import jax.numpy as jnp
from jax import lax
from jax.experimental import pallas as pl
from jax.experimental.pallas import tpu as pltpu
from jax.experimental.pallas import tpu_sc as plsc

NC = 2
NS = 16
NW = NC * NS
L = 16
KE = 80

f32 = jnp.float32
i32 = jnp.int32


def _mesh():
  return plsc.VectorSubcoreMesh(core_axis_name="c", subcore_axis_name="s")


def _fill_rows(ref, nrows, ncols, value):
  v = jnp.full((L,), value, f32)

  def body(k, _):
    for d in range(ncols // L):
      ref[k, pl.ds(d * L, L)] = v
    return 0

  lax.fori_loop(0, nrows, body, 0)


def _sc_gather_deg(t_pad, t2e_pad, dst, npad, d, e):
  rpw = npad // NW
  spw = npad // NS
  epw = e // NW
  nblk = epw // KE

  @functools.partial(
      pl.kernel,
      out_type=(
          jax.ShapeDtypeStruct((npad, d), f32),
          jax.ShapeDtypeStruct((NC, npad, L), f32),
      ),
      mesh=_mesh(),
      compiler_params=pltpu.CompilerParams(needs_layout_passes=False),
      scratch_types=[
          pltpu.VMEM((KE,), i32),
          pltpu.VMEM((KE, d), f32),
          pltpu.VMEM((KE, L), f32),
          pltpu.VMEM((KE, L), f32),
          pltpu.VMEM_SHARED((npad, L), f32),
          pltpu.SemaphoreType.DMA,
      ],
  )
  def k(t_h, t2e_h, dst_h, ee0_h, degp_h, idx_v, rows_v, val_v, stg_v,
        deg_sp, sem):
    c = lax.axis_index("c")
    s = lax.axis_index("s")
    wid = s * NC + c
    _fill_rows(val_v, KE, L, 1.0 / L)
    _fill_rows(stg_v, KE, L, 0.0)
    for j in range(spw // KE):
      pltpu.sync_copy(stg_v, deg_sp.at[pl.ds(s * spw + j * KE, KE)])
    plsc.subcore_barrier()
    for j in range(rpw // KE):
      base = wid * rpw + j * KE
      pltpu.sync_copy(t2e_h.at[pl.ds(base, KE)], idx_v)
      pltpu.async_copy(t_h.at[idx_v], rows_v, sem).wait()
      pltpu.sync_copy(rows_v, ee0_h.at[pl.ds(base, KE)])

    def body(j, _):
      base = wid * epw + j * KE
      pltpu.sync_copy(dst_h.at[pl.ds(base, KE)], idx_v)
      pltpu.sync_copy(val_v, deg_sp.at[idx_v], add=True)
      return 0

    lax.fori_loop(0, nblk, body, 0)
    plsc.subcore_barrier()
    for j in range(spw // KE):
      r = pl.ds(s * spw + j * KE, KE)
      pltpu.sync_copy(deg_sp.at[r], stg_v)
      pltpu.sync_copy(stg_v, degp_h.at[c, r])

  return k(t_pad, t2e_pad, dst)


def _sc_seg_accum(xws, src, dst, npad, d, e):
  spw = npad // NS
  epw = e // NW
  nblk = epw // KE

  @functools.partial(
      pl.kernel,
      out_type=jax.ShapeDtypeStruct((NC, npad, d), f32),
      mesh=_mesh(),
      compiler_params=pltpu.CompilerParams(needs_layout_passes=False),
      scratch_types=[
          pltpu.VMEM((KE,), i32),
          pltpu.VMEM((KE,), i32),
          pltpu.VMEM((KE,), i32),
          pltpu.VMEM((KE,), i32),
          pltpu.VMEM((KE, d), f32),
          pltpu.VMEM((KE, d), f32),
          pltpu.VMEM_SHARED((npad, d), f32),
          pltpu.SemaphoreType.DMA,
      ],
  )
  def k(xws_h, src_h, dst_h, accp_h, sidx_v, didx_v, sidx2_v, didx2_v,
        rows_v, rows2_v, acc_sp, sem):
    c = lax.axis_index("c")
    s = lax.axis_index("s")
    wid = s * NC + c
    _fill_rows(rows_v, KE, d, 0.0)
    for j in range(spw // KE):
      pltpu.sync_copy(rows_v, acc_sp.at[pl.ds(s * spw + j * KE, KE)])
    plsc.subcore_barrier()

    def blk(j, si_v, di_v, rw_v):
      base = wid * epw + j * KE
      pltpu.sync_copy(src_h.at[pl.ds(base, KE)], si_v)
      pltpu.sync_copy(dst_h.at[pl.ds(base, KE)], di_v)
      pltpu.async_copy(xws_h.at[si_v], rw_v, sem).wait()
      pltpu.sync_copy(rw_v, acc_sp.at[di_v], add=True)

    def body(jp, _):
      blk(2 * jp, sidx_v, didx_v, rows_v)
      blk(2 * jp + 1, sidx2_v, didx2_v, rows2_v)
      return 0

    lax.fori_loop(0, nblk // 2, body, 0)
    if nblk % 2:
      blk(nblk - 1, sidx_v, didx_v, rows_v)
    plsc.subcore_barrier()
    for j in range(spw // KE):
      r = pl.ds(s * spw + j * KE, KE)
      pltpu.sync_copy(acc_sp.at[r], rows_v)
      pltpu.sync_copy(rows_v, accp_h.at[c, r])

  return k(xws, src, dst)


def _sc_edge_dots(qr8, kr, e0, e1, npad, d, kq, e2):
  epw = e2 // NW
  nblk = epw // KE

  @functools.partial(
      pl.kernel,
      out_type=(
          jax.ShapeDtypeStruct((e2,), f32),
          jax.ShapeDtypeStruct((NW, L), f32),
      ),
      mesh=_mesh(),
      compiler_params=pltpu.CompilerParams(needs_layout_passes=False),
      scratch_types=[
          pltpu.VMEM((KE,), i32),
          pltpu.VMEM((KE,), i32),
          pltpu.VMEM((KE, d), f32),
          pltpu.VMEM((KE, d), f32),
          pltpu.VMEM((KE,), f32),
          pltpu.VMEM((L,), f32),
          pltpu.SemaphoreType.DMA,
      ],
  )
  def k(q_h, k_h, e0_h, e1_h, dot_h, maxp_h, i0_v, i1_v, qg_v, kg_v, dot_v,
        mx_v, sem):
    c = lax.axis_index("c")
    s = lax.axis_index("s")
    wid = s * NC + c
    iota = lax.iota(i32, L)

    def body(j, mx):
      base = wid * epw + j * KE
      pltpu.sync_copy(e0_h.at[pl.ds(base, KE)], i0_v)
      pltpu.sync_copy(e1_h.at[pl.ds(base, KE)], i1_v)
      cp_q = pltpu.async_copy(q_h.at[i1_v], qg_v, sem)
      cp_k = pltpu.async_copy(k_h.at[i0_v], kg_v, sem)
      cp_q.wait()
      cp_k.wait()
      for jj in range(KE // L):
        rows = jnp.full((L,), jj * L, i32) + iota
        acc = jnp.zeros((L,), f32)
        for dd in range(kq):
          col = jnp.full((L,), dd, i32)
          acc = acc + (plsc.load_gather(qg_v, [rows, col]) *
                       plsc.load_gather(kg_v, [rows, col]))
        dot_v[pl.ds(jj * L, L)] = acc
        mx = jnp.maximum(mx, acc)
      pltpu.sync_copy(dot_v, dot_h.at[pl.ds(base, KE)])
      return mx

    mx = lax.fori_loop(0, nblk, body, jnp.full((L,), -jnp.inf, f32))
    mx_v[...] = mx
    pltpu.sync_copy(mx_v, maxp_h.at[wid])

  return k(qr8, kr, e0, e1)


def _sc_softmax_accum(gcn, dots, maxp, e0, e1, npad, d, e2):
  spw = npad // NS
  epw = e2 // NW
  nblk = epw // KE

  @functools.partial(
      pl.kernel,
      out_type=jax.ShapeDtypeStruct((NC, npad, d), f32),
      mesh=_mesh(),
      compiler_params=pltpu.CompilerParams(needs_layout_passes=False),
      scratch_types=[
          [pltpu.VMEM((KE,), i32), pltpu.VMEM((KE,), i32)],
          [pltpu.VMEM((KE,), i32), pltpu.VMEM((KE,), i32)],
          [pltpu.VMEM((KE, d), f32), pltpu.VMEM((KE, d), f32)],
          [pltpu.VMEM((KE,), f32), pltpu.VMEM((KE,), f32)],
          pltpu.VMEM((KE,), f32),
          pltpu.VMEM((NW, L), f32),
          pltpu.VMEM_SHARED((npad, d), f32),
          pltpu.SemaphoreType.DMA,
      ],
  )
  def k(gcn_h, dot_h, maxp_h, e0_h, e1_h, nump_h, i0_b, i1_b, rows_b,
        dot_b, ex_v, mxb_v, num_sp, sem):
    c = lax.axis_index("c")
    s = lax.axis_index("s")
    wid = s * NC + c
    pltpu.sync_copy(maxp_h, mxb_v)
    m = mxb_v[0, :]
    for r in range(1, NW):
      m = jnp.maximum(m, mxb_v[r, :])
    iota = lax.iota(i32, L)
    for sh in (8, 4, 2, 1):
      m = jnp.maximum(m, m[iota ^ sh])
    ms = m
    rows_v = rows_b[0]
    _fill_rows(rows_v, KE, d, 0.0)
    for j in range(spw // KE):
      pltpu.sync_copy(rows_v, num_sp.at[pl.ds(s * spw + j * KE, KE)])
    plsc.subcore_barrier()

    def blk(j, i0_v, i1_v, rw_v, dt_v):
      base = wid * epw + j * KE
      pltpu.sync_copy(e0_h.at[pl.ds(base, KE)], i0_v)
      pltpu.sync_copy(e1_h.at[pl.ds(base, KE)], i1_v)
      pltpu.sync_copy(dot_h.at[pl.ds(base, KE)], dt_v)
      pltpu.async_copy(gcn_h.at[i0_v], rw_v, sem).wait()
      for jj in range(KE // L):
        ex_v[pl.ds(jj * L, L)] = jnp.exp(dt_v[pl.ds(jj * L, L)] - ms)

      def scale(kk, _):
        exk = plsc.load_gather(ex_v, [jnp.full((L,), kk, i32)])
        for dd in range(d // L):
          rw_v[kk, pl.ds(dd * L, L)] = rw_v[kk, pl.ds(dd * L, L)] * exk
        return 0

      lax.fori_loop(0, KE, scale, 0)
      pltpu.sync_copy(rw_v, num_sp.at[i1_v], add=True)

    def body(jp, _):
      blk(2 * jp, i0_b[0], i1_b[0], rows_b[0], dot_b[0])
      blk(2 * jp + 1, i0_b[1], i1_b[1], rows_b[1], dot_b[1])
      return 0

    lax.fori_loop(0, nblk // 2, body, 0)
    if nblk % 2:
      blk(nblk - 1, i0_b[0], i1_b[0], rows_b[0], dot_b[0])
    plsc.subcore_barrier()
    for j in range(spw // KE):
      r = pl.ds(s * spw + j * KE, KE)
      pltpu.sync_copy(num_sp.at[r], rows_v)
      pltpu.sync_copy(rows_v, nump_h.at[c, r])

  return k(gcn, dots, maxp, e0, e1)


def _sc_softmax_denom(dots, maxp, e1, npad, e2):
  spw = npad // NS
  epw = e2 // NW
  nblk = epw // KE

  @functools.partial(
      pl.kernel,
      out_type=jax.ShapeDtypeStruct((NC, npad, L), f32),
      mesh=_mesh(),
      compiler_params=pltpu.CompilerParams(needs_layout_passes=False),
      scratch_types=[
          [pltpu.VMEM((KE,), i32), pltpu.VMEM((KE,), i32)],
          [pltpu.VMEM((KE,), f32), pltpu.VMEM((KE,), f32)],
          [pltpu.VMEM((KE, L), f32), pltpu.VMEM((KE, L), f32)],
          pltpu.VMEM((KE,), f32),
          pltpu.VMEM((NW, L), f32),
          pltpu.VMEM_SHARED((npad, L), f32),
          pltpu.SemaphoreType.DMA,
      ],
  )
  def k(dot_h, maxp_h, e1_h, denp_h, i1_b, dot_b, exr_b, ex_v, mxb_v,
        den_sp, sem):
    c = lax.axis_index("c")
    s = lax.axis_index("s")
    wid = s * NC + c
    pltpu.sync_copy(maxp_h, mxb_v)
    m = mxb_v[0, :]
    for r in range(1, NW):
      m = jnp.maximum(m, mxb_v[r, :])
    iota = lax.iota(i32, L)
    for sh in (8, 4, 2, 1):
      m = jnp.maximum(m, m[iota ^ sh])
    ms = m
    exr_v = exr_b[0]
    _fill_rows(exr_v, KE, L, 0.0)
    for j in range(spw // KE):
      pltpu.sync_copy(exr_v, den_sp.at[pl.ds(s * spw + j * KE, KE)])
    plsc.subcore_barrier()

    def blk(j, i1_v, dt_v, xr_v):
      base = wid * epw + j * KE
      pltpu.sync_copy(e1_h.at[pl.ds(base, KE)], i1_v)
      pltpu.sync_copy(dot_h.at[pl.ds(base, KE)], dt_v)
      for jj in range(KE // L):
        ex_v[pl.ds(jj * L, L)] = jnp.exp(dt_v[pl.ds(jj * L, L)] - ms)

      def fill(kk, _):
        exk = plsc.load_gather(ex_v, [jnp.full((L,), kk, i32)])
        xr_v[kk, :] = exk * (1.0 / L)
        return 0

      lax.fori_loop(0, KE, fill, 0)
      pltpu.sync_copy(xr_v, den_sp.at[i1_v], add=True)

    def body(jp, _):
      blk(2 * jp, i1_b[0], dot_b[0], exr_b[0])
      blk(2 * jp + 1, i1_b[1], dot_b[1], exr_b[1])
      return 0

    lax.fori_loop(0, nblk // 2, body, 0)
    if nblk % 2:
      blk(nblk - 1, i1_b[0], dot_b[0], exr_b[0])
    plsc.subcore_barrier()
    for j in range(spw // KE):
      r = pl.ds(s * spw + j * KE, KE)
      pltpu.sync_copy(den_sp.at[r], exr_v)
      pltpu.sync_copy(exr_v, denp_h.at[c, r])

  return k(dots, maxp, e1)


_BLK = 512


def _row_spec(bd):
  return pl.BlockSpec((_BLK, bd), lambda i: (i, 0))


def _const_spec(shape):
  return pl.BlockSpec(shape, lambda i: (0, 0))


def _tc_pre(t_pad, ee0, degp, w_gcn, w_q, b_q, npad, d, kq):
  def body(t_r, e_r, d0_r, d1_r, wg_r, wq_r, bq_r, xws_o, qr8_o):
    deg = jnp.sum(d0_r[...] + d1_r[...], axis=1, keepdims=True) + 1.0
    dis = lax.rsqrt(deg)
    xws_o[...] = dis * jnp.dot(e_r[...], wg_r[...],
                               preferred_element_type=f32)
    qr8_o[...] = (jnp.dot(t_r[...], wq_r[...],
                          preferred_element_type=f32) + bq_r[...]) * 0.125

  return pl.pallas_call(
      body,
      grid=(npad // _BLK,),
      in_specs=[
          _row_spec(d), _row_spec(d), _row_spec(L), _row_spec(L),
          _const_spec((d, d)), _const_spec((d, d)), _const_spec((1, d)),
      ],
      out_specs=[_row_spec(d), _row_spec(d)],
      out_shape=[
          jax.ShapeDtypeStruct((npad, d), f32),
          jax.ShapeDtypeStruct((npad, d), f32),
      ],
  )(t_pad, ee0, degp[0], degp[1], w_gcn, w_q, b_q)


def _tc_mid(accp, xws, degp, w_k, b_k, b_g, npad, d, kq):
  def body(a0_r, a1_r, x_r, d0_r, d1_r, wk_r, bk_r, bg_r, gcn_o, kr_o):
    deg = jnp.sum(d0_r[...] + d1_r[...], axis=1, keepdims=True) + 1.0
    dis = lax.rsqrt(deg)
    g = dis * (a0_r[...] + a1_r[...] + x_r[...]) + bg_r[...]
    gcn_o[...] = g
    kr_o[...] = jnp.dot(g, wk_r[...], preferred_element_type=f32) + bk_r[...]

  return pl.pallas_call(
      body,
      grid=(npad // _BLK,),
      in_specs=[
          _row_spec(d), _row_spec(d), _row_spec(d), _row_spec(L),
          _row_spec(L), _const_spec((d, d)), _const_spec((1, d)),
          _const_spec((1, d)),
      ],
      out_specs=[_row_spec(d), _row_spec(d)],
      out_shape=[
          jax.ShapeDtypeStruct((npad, d), f32),
          jax.ShapeDtypeStruct((npad, d), f32),
      ],
  )(accp[0], accp[1], xws, degp[0], degp[1], w_k, b_k, b_g)


def _tc_post(t_pad, nump, denp, w_lin, b_lin, gates, npad, d):
  def body(t_r, n0_r, n1_r, d0_r, d1_r, wl_r, bl_r, g_r, out_o):
    ga = jnp.tanh(g_r[0, 0])
    gb = jnp.tanh(g_r[0, 1])
    den = jnp.sum(d0_r[...] + d1_r[...], axis=1, keepdims=True)
    attn = (n0_r[...] + n1_r[...]) / (den + 1e-16)
    new = t_r[...] + ga * attn
    out_o[...] = new + gb * (jnp.dot(new, wl_r[...],
                                     preferred_element_type=f32) + bl_r[...])

  return pl.pallas_call(
      body,
      grid=(npad // _BLK,),
      in_specs=[
          _row_spec(d), _row_spec(d), _row_spec(d), _row_spec(L),
          _row_spec(L), _const_spec((d, d)), _const_spec((1, d)),
          _const_spec((1, 2)),
      ],
      out_specs=_row_spec(d),
      out_shape=jax.ShapeDtypeStruct((npad, d), f32),
  )(t_pad, nump[0], nump[1], denp[0], denp[1], w_lin, b_lin, gates)


def kernel(token_embeddings, tokens2edges, edge_index, edges2tokens, W_gcn,
           b_gcn, W_key, b_key, W_query, b_query, W_lin, b_lin, gate_a,
           gate_b):
  bsz, t_len, d = token_embeddings.shape
  n = tokens2edges.shape[1]
  e = edge_index.shape[2]
  e2 = edges2tokens.shape[2]
  kq = W_key.shape[1]
  assert t_len == n
  npad = ((n + NW * KE - 1) // (NW * KE)) * (NW * KE)
  assert e % (NW * KE) == 0 and e2 % (NW * KE) == 0

  b_gcn2 = b_gcn.reshape(1, d)
  w_key_p = jnp.pad(W_key, ((0, 0), (0, d - kq)))
  b_key2 = jnp.pad(b_key.reshape(1, kq), ((0, 0), (0, d - kq)))
  w_query_p = jnp.pad(W_query, ((0, 0), (0, d - kq)))
  b_query2 = jnp.pad(b_query.reshape(1, kq), ((0, 0), (0, d - kq)))
  b_lin2 = b_lin.reshape(1, d)
  gates = jnp.concatenate([gate_a, gate_b]).reshape(1, 2)

  outs = []
  for b in range(bsz):
    t_pad = jnp.pad(token_embeddings[b], ((0, npad - t_len), (0, 0)))
    t2e_pad = jnp.pad(tokens2edges[b], (0, npad - n))
    src = edge_index[b, 0]
    dst = edge_index[b, 1]
    e0 = edges2tokens[b, 0]
    e1 = edges2tokens[b, 1]

    ee0, degp = _sc_gather_deg(t_pad, t2e_pad, dst, npad, d, e)
    xws, qr8 = _tc_pre(t_pad, ee0, degp, W_gcn, w_query_p, b_query2, npad,
                       d, kq)
    accp = _sc_seg_accum(xws, src, dst, npad, d, e)
    gcn, kr = _tc_mid(accp, xws, degp, w_key_p, b_key2, b_gcn2, npad, d, kq)
    dots, maxp = _sc_edge_dots(qr8, kr, e0, e1, npad, d, kq, e2)
    nump = _sc_softmax_accum(gcn, dots, maxp, e0, e1, npad, d, e2)
    denp = _sc_softmax_denom(dots, maxp, e1, npad, e2)
    out = _tc_post(t_pad, nump, denp, W_lin, b_lin2, gates, npad, d)
    outs.append(out[:t_len])
  return jnp.stack(outs, axis=0)


# speedup vs baseline: 1.1420x; 1.1420x over previous
import functools

import jax
import jax.numpy as jnp
from jax import lax
from jax.experimental import pallas as pl
from jax.experimental.pallas import tpu as pltpu
from jax.experimental.pallas import tpu_sc as plsc

NC = 2
NS = 16
NW = NC * NS
L = 16
KE = 80

f32 = jnp.float32
i32 = jnp.int32


def _mesh():
  return plsc.VectorSubcoreMesh(core_axis_name="c", subcore_axis_name="s")


def _fill_rows(ref, nrows, ncols, value):
  v = jnp.full((L,), value, f32)

  def body(k, _):
    for d in range(ncols // L):
      ref[k, pl.ds(d * L, L)] = v
    return 0

  lax.fori_loop(0, nrows, body, 0)


def _sc_gather_deg(t_pad, t2e_pad, dst, npad, d, e):
  rpw = npad // NW
  spw = npad // NS
  epw = e // NW
  nblk = epw // KE

  @functools.partial(
      pl.kernel,
      out_type=(
          jax.ShapeDtypeStruct((npad, d), f32),
          jax.ShapeDtypeStruct((NC, npad, L), f32),
      ),
      mesh=_mesh(),
      compiler_params=pltpu.CompilerParams(needs_layout_passes=False),
      scratch_types=[
          pltpu.VMEM((KE,), i32),
          pltpu.VMEM((KE, d), f32),
          pltpu.VMEM((KE, L), f32),
          pltpu.VMEM((KE, L), f32),
          pltpu.VMEM_SHARED((npad, L), f32),
          pltpu.SemaphoreType.DMA,
      ],
  )
  def k(t_h, t2e_h, dst_h, ee0_h, degp_h, idx_v, rows_v, val_v, stg_v,
        deg_sp, sem):
    c = lax.axis_index("c")
    s = lax.axis_index("s")
    wid = s * NC + c
    _fill_rows(val_v, KE, L, 1.0 / L)
    _fill_rows(stg_v, KE, L, 0.0)
    for j in range(spw // KE):
      pltpu.sync_copy(stg_v, deg_sp.at[pl.ds(s * spw + j * KE, KE)])
    plsc.subcore_barrier()
    for j in range(rpw // KE):
      base = wid * rpw + j * KE
      pltpu.sync_copy(t2e_h.at[pl.ds(base, KE)], idx_v)
      pltpu.async_copy(t_h.at[idx_v], rows_v, sem).wait()
      pltpu.sync_copy(rows_v, ee0_h.at[pl.ds(base, KE)])

    def body(j, _):
      base = wid * epw + j * KE
      pltpu.sync_copy(dst_h.at[pl.ds(base, KE)], idx_v)
      pltpu.sync_copy(val_v, deg_sp.at[idx_v], add=True)
      return 0

    lax.fori_loop(0, nblk, body, 0)
    plsc.subcore_barrier()
    for j in range(spw // KE):
      r = pl.ds(s * spw + j * KE, KE)
      pltpu.sync_copy(deg_sp.at[r], stg_v)
      pltpu.sync_copy(stg_v, degp_h.at[c, r])

  return k(t_pad, t2e_pad, dst)


def _sc_seg_accum(xws, src, dst, npad, d, e):
  spw = npad // NS
  epw = e // NW
  nblk = epw // KE

  @functools.partial(
      pl.kernel,
      out_type=jax.ShapeDtypeStruct((NC, npad, d), f32),
      mesh=_mesh(),
      compiler_params=pltpu.CompilerParams(needs_layout_passes=False),
      scratch_types=[
          pltpu.VMEM((KE,), i32),
          pltpu.VMEM((KE,), i32),
          pltpu.VMEM((KE,), i32),
          pltpu.VMEM((KE,), i32),
          pltpu.VMEM((KE, d), f32),
          pltpu.VMEM((KE, d), f32),
          pltpu.VMEM_SHARED((npad, d), f32),
          pltpu.SemaphoreType.DMA,
          pltpu.SemaphoreType.DMA,
      ],
  )
  def k(xws_h, src_h, dst_h, accp_h, sidx_v, didx_v, sidx2_v, didx2_v,
        rows_v, rows2_v, acc_sp, sem, semi):
    c = lax.axis_index("c")
    s = lax.axis_index("s")
    wid = s * NC + c
    _fill_rows(rows_v, KE, d, 0.0)
    for j in range(spw // KE):
      pltpu.sync_copy(rows_v, acc_sp.at[pl.ds(s * spw + j * KE, KE)])
    plsc.subcore_barrier()

    def blk(j, si_v, di_v, rw_v):
      base = wid * epw + j * KE
      c_si = pltpu.async_copy(src_h.at[pl.ds(base, KE)], si_v, semi)
      c_di = pltpu.async_copy(dst_h.at[pl.ds(base, KE)], di_v, semi)
      c_si.wait()
      c_g = pltpu.async_copy(xws_h.at[si_v], rw_v, sem)
      c_di.wait()
      c_g.wait()
      pltpu.sync_copy(rw_v, acc_sp.at[di_v], add=True)

    def body(jp, _):
      blk(2 * jp, sidx_v, didx_v, rows_v)
      blk(2 * jp + 1, sidx2_v, didx2_v, rows2_v)
      return 0

    lax.fori_loop(0, nblk // 2, body, 0)
    if nblk % 2:
      blk(nblk - 1, sidx_v, didx_v, rows_v)
    plsc.subcore_barrier()
    for j in range(spw // KE):
      r = pl.ds(s * spw + j * KE, KE)
      pltpu.sync_copy(acc_sp.at[r], rows_v)
      pltpu.sync_copy(rows_v, accp_h.at[c, r])

  return k(xws, src, dst)


def _sc_edge_dots(qr8, kr, e0, e1, npad, d, kq, e2):
  epw = e2 // NW
  nblk = epw // KE

  @functools.partial(
      pl.kernel,
      out_type=(
          jax.ShapeDtypeStruct((e2,), f32),
          jax.ShapeDtypeStruct((NW, L), f32),
      ),
      mesh=_mesh(),
      compiler_params=pltpu.CompilerParams(needs_layout_passes=False),
      scratch_types=[
          pltpu.VMEM((KE,), i32),
          pltpu.VMEM((KE,), i32),
          pltpu.VMEM((KE, d), f32),
          pltpu.VMEM((KE, d), f32),
          pltpu.VMEM((KE,), f32),
          pltpu.VMEM((L,), f32),
          pltpu.SemaphoreType.DMA,
      ],
  )
  def k(q_h, k_h, e0_h, e1_h, dot_h, maxp_h, i0_v, i1_v, qg_v, kg_v, dot_v,
        mx_v, sem):
    c = lax.axis_index("c")
    s = lax.axis_index("s")
    wid = s * NC + c
    iota = lax.iota(i32, L)

    def body(j, mx):
      base = wid * epw + j * KE
      c_i0 = pltpu.async_copy(e0_h.at[pl.ds(base, KE)], i0_v, sem)
      c_i1 = pltpu.async_copy(e1_h.at[pl.ds(base, KE)], i1_v, sem)
      c_i0.wait()
      c_i1.wait()
      cp_q = pltpu.async_copy(q_h.at[i1_v], qg_v, sem)
      cp_k = pltpu.async_copy(k_h.at[i0_v], kg_v, sem)
      cp_q.wait()
      cp_k.wait()
      for jj in range(KE // L):
        rows = jnp.full((L,), jj * L, i32) + iota
        acc = jnp.zeros((L,), f32)
        for dd in range(kq):
          col = jnp.full((L,), dd, i32)
          acc = acc + (plsc.load_gather(qg_v, [rows, col]) *
                       plsc.load_gather(kg_v, [rows, col]))
        dot_v[pl.ds(jj * L, L)] = acc
        mx = jnp.maximum(mx, acc)
      pltpu.sync_copy(dot_v, dot_h.at[pl.ds(base, KE)])
      return mx

    mx = lax.fori_loop(0, nblk, body, jnp.full((L,), -jnp.inf, f32))
    mx_v[...] = mx
    pltpu.sync_copy(mx_v, maxp_h.at[wid])

  return k(qr8, kr, e0, e1)


def _sc_softmax_accum(gcn, dots, maxp, e0, e1, npad, d, e2):
  spw = npad // NS
  epw = e2 // NW
  nblk = epw // KE

  @functools.partial(
      pl.kernel,
      out_type=jax.ShapeDtypeStruct((NC, npad, d), f32),
      mesh=_mesh(),
      compiler_params=pltpu.CompilerParams(needs_layout_passes=False),
      scratch_types=[
          [pltpu.VMEM((KE,), i32), pltpu.VMEM((KE,), i32)],
          [pltpu.VMEM((KE,), i32), pltpu.VMEM((KE,), i32)],
          [pltpu.VMEM((KE, d), f32), pltpu.VMEM((KE, d), f32)],
          [pltpu.VMEM((KE,), f32), pltpu.VMEM((KE,), f32)],
          pltpu.VMEM((KE,), f32),
          pltpu.VMEM((NW, L), f32),
          pltpu.VMEM_SHARED((npad, d), f32),
          pltpu.SemaphoreType.DMA,
          pltpu.SemaphoreType.DMA,
      ],
  )
  def k(gcn_h, dot_h, maxp_h, e0_h, e1_h, nump_h, i0_b, i1_b, rows_b,
        dot_b, ex_v, mxb_v, num_sp, sem, semi):
    c = lax.axis_index("c")
    s = lax.axis_index("s")
    wid = s * NC + c
    pltpu.sync_copy(maxp_h, mxb_v)
    m = mxb_v[0, :]
    for r in range(1, NW):
      m = jnp.maximum(m, mxb_v[r, :])
    iota = lax.iota(i32, L)
    for sh in (8, 4, 2, 1):
      m = jnp.maximum(m, m[iota ^ sh])
    ms = m
    rows_v = rows_b[0]
    _fill_rows(rows_v, KE, d, 0.0)
    for j in range(spw // KE):
      pltpu.sync_copy(rows_v, num_sp.at[pl.ds(s * spw + j * KE, KE)])
    plsc.subcore_barrier()

    def blk(j, i0_v, i1_v, rw_v, dt_v):
      base = wid * epw + j * KE
      c_i0 = pltpu.async_copy(e0_h.at[pl.ds(base, KE)], i0_v, semi)
      c_i1 = pltpu.async_copy(e1_h.at[pl.ds(base, KE)], i1_v, semi)
      c_dt = pltpu.async_copy(dot_h.at[pl.ds(base, KE)], dt_v, semi)
      c_i0.wait()
      c_g = pltpu.async_copy(gcn_h.at[i0_v], rw_v, sem)
      c_i1.wait()
      c_dt.wait()
      c_g.wait()
      for jj in range(KE // L):
        ex_v[pl.ds(jj * L, L)] = jnp.exp(dt_v[pl.ds(jj * L, L)] - ms)

      def scale(kk, _):
        exk = plsc.load_gather(ex_v, [jnp.full((L,), kk, i32)])
        for dd in range(d // L):
          rw_v[kk, pl.ds(dd * L, L)] = rw_v[kk, pl.ds(dd * L, L)] * exk
        return 0

      lax.fori_loop(0, KE, scale, 0)
      pltpu.sync_copy(rw_v, num_sp.at[i1_v], add=True)

    def body(jp, _):
      blk(2 * jp, i0_b[0], i1_b[0], rows_b[0], dot_b[0])
      blk(2 * jp + 1, i0_b[1], i1_b[1], rows_b[1], dot_b[1])
      return 0

    lax.fori_loop(0, nblk // 2, body, 0)
    if nblk % 2:
      blk(nblk - 1, i0_b[0], i1_b[0], rows_b[0], dot_b[0])
    plsc.subcore_barrier()
    for j in range(spw // KE):
      r = pl.ds(s * spw + j * KE, KE)
      pltpu.sync_copy(num_sp.at[r], rows_v)
      pltpu.sync_copy(rows_v, nump_h.at[c, r])

  return k(gcn, dots, maxp, e0, e1)


def _sc_softmax_denom(dots, maxp, e1, npad, e2):
  spw = npad // NS
  epw = e2 // NW
  nblk = epw // KE

  @functools.partial(
      pl.kernel,
      out_type=jax.ShapeDtypeStruct((NC, npad, L), f32),
      mesh=_mesh(),
      compiler_params=pltpu.CompilerParams(needs_layout_passes=False),
      scratch_types=[
          [pltpu.VMEM((KE,), i32), pltpu.VMEM((KE,), i32)],
          [pltpu.VMEM((KE,), f32), pltpu.VMEM((KE,), f32)],
          [pltpu.VMEM((KE, L), f32), pltpu.VMEM((KE, L), f32)],
          pltpu.VMEM((KE,), f32),
          pltpu.VMEM((NW, L), f32),
          pltpu.VMEM_SHARED((npad, L), f32),
          pltpu.SemaphoreType.DMA,
      ],
  )
  def k(dot_h, maxp_h, e1_h, denp_h, i1_b, dot_b, exr_b, ex_v, mxb_v,
        den_sp, sem):
    c = lax.axis_index("c")
    s = lax.axis_index("s")
    wid = s * NC + c
    pltpu.sync_copy(maxp_h, mxb_v)
    m = mxb_v[0, :]
    for r in range(1, NW):
      m = jnp.maximum(m, mxb_v[r, :])
    iota = lax.iota(i32, L)
    for sh in (8, 4, 2, 1):
      m = jnp.maximum(m, m[iota ^ sh])
    ms = m
    exr_v = exr_b[0]
    _fill_rows(exr_v, KE, L, 0.0)
    for j in range(spw // KE):
      pltpu.sync_copy(exr_v, den_sp.at[pl.ds(s * spw + j * KE, KE)])
    plsc.subcore_barrier()

    def blk(j, i1_v, dt_v, xr_v):
      base = wid * epw + j * KE
      c_i1 = pltpu.async_copy(e1_h.at[pl.ds(base, KE)], i1_v, sem)
      c_dt = pltpu.async_copy(dot_h.at[pl.ds(base, KE)], dt_v, sem)
      c_i1.wait()
      c_dt.wait()
      for jj in range(KE // L):
        ex_v[pl.ds(jj * L, L)] = jnp.exp(dt_v[pl.ds(jj * L, L)] - ms)

      def fill(kk, _):
        exk = plsc.load_gather(ex_v, [jnp.full((L,), kk, i32)])
        xr_v[kk, :] = exk * (1.0 / L)
        return 0

      lax.fori_loop(0, KE, fill, 0)
      pltpu.sync_copy(xr_v, den_sp.at[i1_v], add=True)

    def body(jp, _):
      blk(2 * jp, i1_b[0], dot_b[0], exr_b[0])
      blk(2 * jp + 1, i1_b[1], dot_b[1], exr_b[1])
      return 0

    lax.fori_loop(0, nblk // 2, body, 0)
    if nblk % 2:
      blk(nblk - 1, i1_b[0], dot_b[0], exr_b[0])
    plsc.subcore_barrier()
    for j in range(spw // KE):
      r = pl.ds(s * spw + j * KE, KE)
      pltpu.sync_copy(den_sp.at[r], exr_v)
      pltpu.sync_copy(exr_v, denp_h.at[c, r])

  return k(dots, maxp, e1)


_BLK = 512


def _row_spec(bd):
  return pl.BlockSpec((_BLK, bd), lambda i: (i, 0))


def _const_spec(shape):
  return pl.BlockSpec(shape, lambda i: (0, 0))


def _tc_pre(t_pad, ee0, degp, w_gcn, w_q, b_q, npad, d, kq):
  def body(t_r, e_r, d0_r, d1_r, wg_r, wq_r, bq_r, xws_o, qr8_o):
    deg = jnp.sum(d0_r[...] + d1_r[...], axis=1, keepdims=True) + 1.0
    dis = lax.rsqrt(deg)
    xws_o[...] = dis * jnp.dot(e_r[...], wg_r[...],
                               preferred_element_type=f32)
    qr8_o[...] = (jnp.dot(t_r[...], wq_r[...],
                          preferred_element_type=f32) + bq_r[...]) * 0.125

  return pl.pallas_call(
      body,
      grid=(npad // _BLK,),
      in_specs=[
          _row_spec(d), _row_spec(d), _row_spec(L), _row_spec(L),
          _const_spec((d, d)), _const_spec((d, d)), _const_spec((1, d)),
      ],
      out_specs=[_row_spec(d), _row_spec(d)],
      out_shape=[
          jax.ShapeDtypeStruct((npad, d), f32),
          jax.ShapeDtypeStruct((npad, d), f32),
      ],
  )(t_pad, ee0, degp[0], degp[1], w_gcn, w_q, b_q)


def _tc_mid(accp, xws, degp, w_k, b_k, b_g, npad, d, kq):
  def body(a0_r, a1_r, x_r, d0_r, d1_r, wk_r, bk_r, bg_r, gcn_o, kr_o):
    deg = jnp.sum(d0_r[...] + d1_r[...], axis=1, keepdims=True) + 1.0
    dis = lax.rsqrt(deg)
    g = dis * (a0_r[...] + a1_r[...] + x_r[...]) + bg_r[...]
    gcn_o[...] = g
    kr_o[...] = jnp.dot(g, wk_r[...], preferred_element_type=f32) + bk_r[...]

  return pl.pallas_call(
      body,
      grid=(npad // _BLK,),
      in_specs=[
          _row_spec(d), _row_spec(d), _row_spec(d), _row_spec(L),
          _row_spec(L), _const_spec((d, d)), _const_spec((1, d)),
          _const_spec((1, d)),
      ],
      out_specs=[_row_spec(d), _row_spec(d)],
      out_shape=[
          jax.ShapeDtypeStruct((npad, d), f32),
          jax.ShapeDtypeStruct((npad, d), f32),
      ],
  )(accp[0], accp[1], xws, degp[0], degp[1], w_k, b_k, b_g)


def _tc_post(t_pad, nump, denp, w_lin, b_lin, gates, npad, d):
  def body(t_r, n0_r, n1_r, d0_r, d1_r, wl_r, bl_r, g_r, out_o):
    ga = jnp.tanh(g_r[0, 0])
    gb = jnp.tanh(g_r[0, 1])
    den = jnp.sum(d0_r[...] + d1_r[...], axis=1, keepdims=True)
    attn = (n0_r[...] + n1_r[...]) / (den + 1e-16)
    new = t_r[...] + ga * attn
    out_o[...] = new + gb * (jnp.dot(new, wl_r[...],
                                     preferred_element_type=f32) + bl_r[...])

  return pl.pallas_call(
      body,
      grid=(npad // _BLK,),
      in_specs=[
          _row_spec(d), _row_spec(d), _row_spec(d), _row_spec(L),
          _row_spec(L), _const_spec((d, d)), _const_spec((1, d)),
          _const_spec((1, 2)),
      ],
      out_specs=_row_spec(d),
      out_shape=jax.ShapeDtypeStruct((npad, d), f32),
  )(t_pad, nump[0], nump[1], denp[0], denp[1], w_lin, b_lin, gates)


def kernel(token_embeddings, tokens2edges, edge_index, edges2tokens, W_gcn,
           b_gcn, W_key, b_key, W_query, b_query, W_lin, b_lin, gate_a,
           gate_b):
  bsz, t_len, d = token_embeddings.shape
  n = tokens2edges.shape[1]
  e = edge_index.shape[2]
  e2 = edges2tokens.shape[2]
  kq = W_key.shape[1]
  assert t_len == n
  npad = ((n + NW * KE - 1) // (NW * KE)) * (NW * KE)
  assert e % (NW * KE) == 0 and e2 % (NW * KE) == 0

  b_gcn2 = b_gcn.reshape(1, d)
  w_key_p = jnp.pad(W_key, ((0, 0), (0, d - kq)))
  b_key2 = jnp.pad(b_key.reshape(1, kq), ((0, 0), (0, d - kq)))
  w_query_p = jnp.pad(W_query, ((0, 0), (0, d - kq)))
  b_query2 = jnp.pad(b_query.reshape(1, kq), ((0, 0), (0, d - kq)))
  b_lin2 = b_lin.reshape(1, d)
  gates = jnp.concatenate([gate_a, gate_b]).reshape(1, 2)

  outs = []
  for b in range(bsz):
    t_pad = jnp.pad(token_embeddings[b], ((0, npad - t_len), (0, 0)))
    t2e_pad = jnp.pad(tokens2edges[b], (0, npad - n))
    src = edge_index[b, 0]
    dst = edge_index[b, 1]
    e0 = edges2tokens[b, 0]
    e1 = edges2tokens[b, 1]

    ee0, degp = _sc_gather_deg(t_pad, t2e_pad, dst, npad, d, e)
    xws, qr8 = _tc_pre(t_pad, ee0, degp, W_gcn, w_query_p, b_query2, npad,
                       d, kq)
    accp = _sc_seg_accum(xws, src, dst, npad, d, e)
    gcn, kr = _tc_mid(accp, xws, degp, w_key_p, b_key2, b_gcn2, npad, d, kq)
    dots, maxp = _sc_edge_dots(qr8, kr, e0, e1, npad, d, kq, e2)
    nump = _sc_softmax_accum(gcn, dots, maxp, e0, e1, npad, d, e2)
    denp = _sc_softmax_denom(dots, maxp, e1, npad, e2)
    out = _tc_post(t_pad, nump, denp, W_lin, b_lin2, gates, npad, d)
    outs.append(out[:t_len])
  return jnp.stack(outs, axis=0)
